# baseline (speedup 1.0000x reference)
import jax
import jax.numpy as jnp
from jax.experimental import pallas as pl
from jax.experimental.pallas import tpu as pltpu

_NBUF = 3
_MAX_ROWS = 2048
_CHUNKS = [512, 512, 1024] + [2048] * 14 + [1024, 512, 512]
_OFFS = [sum(_CHUNKS[:i]) for i in range(len(_CHUNKS))]


def _body(x_hbm, o_hbm, buf, in_sems, out_sems):
    n = len(_CHUNKS)

    def in_copy(i):
        b = i % _NBUF
        return pltpu.make_async_copy(
            x_hbm.at[pl.ds(_OFFS[i], _CHUNKS[i]), :],
            buf.at[b, pl.ds(0, _CHUNKS[i]), :], in_sems.at[b])

    def out_copy(i):
        b = i % _NBUF
        return pltpu.make_async_copy(
            buf.at[b, pl.ds(0, _CHUNKS[i]), :],
            o_hbm.at[pl.ds(_OFFS[i], _CHUNKS[i]), :], out_sems.at[b])

    in_copy(0).start()
    in_copy(1).start()

    for i in range(n):
        b = i % _NBUF
        if i + 2 < n:
            if i + 2 >= _NBUF:
                out_copy(i + 2 - _NBUF).wait()
            in_copy(i + 2).start()
        in_copy(i).wait()
        sl = pl.ds(0, _CHUNKS[i])
        buf[b, sl, :] = jnp.clip(buf[b, sl, :], 0.0, 1.0)
        out_copy(i).start()

    for i in range(n - _NBUF, n):
        out_copy(i).wait()


def kernel(x):
    n_rows, n_cols = x.shape
    assert n_rows == sum(_CHUNKS)
    return pl.pallas_call(
        _body,
        in_specs=[pl.BlockSpec(memory_space=pl.ANY)],
        out_specs=pl.BlockSpec(memory_space=pl.ANY),
        out_shape=jax.ShapeDtypeStruct((n_rows, n_cols), x.dtype),
        scratch_shapes=[
            pltpu.VMEM((_NBUF, _MAX_ROWS, n_cols), x.dtype),
            pltpu.SemaphoreType.DMA((_NBUF,)),
            pltpu.SemaphoreType.DMA((_NBUF,)),
        ],
        compiler_params=pltpu.CompilerParams(
            vmem_limit_bytes=60 * 1024 * 1024,
        ),
    )(x)


# speedup vs baseline: 1.0039x; 1.0039x over previous
import jax
import jax.numpy as jnp
from jax.experimental import pallas as pl
from jax.experimental.pallas import tpu as pltpu

_CH_ROWS = 2048
_NBUF = 3
_PF = 1


def _body(x_hbm, o_hbm, buf, in_sems, out_sems):
    n_rows = x_hbm.shape[0]
    n = n_rows // _CH_ROWS

    def in_copy(i):
        b = i % _NBUF
        return pltpu.make_async_copy(
            x_hbm.at[pl.ds(i * _CH_ROWS, _CH_ROWS), :],
            buf.at[b], in_sems.at[b])

    def out_copy(i):
        b = i % _NBUF
        return pltpu.make_async_copy(
            buf.at[b],
            o_hbm.at[pl.ds(i * _CH_ROWS, _CH_ROWS), :], out_sems.at[b])

    for i in range(_PF):
        in_copy(i).start()

    for i in range(n):
        b = i % _NBUF
        if i + _PF < n:
            if i + _PF >= _NBUF:
                out_copy(i + _PF - _NBUF).wait()
            in_copy(i + _PF).start()
        in_copy(i).wait()
        buf[b] = jnp.clip(buf[b], 0.0, 1.0)
        out_copy(i).start()

    for i in range(n - _NBUF, n):
        out_copy(i).wait()


def kernel(x):
    n_rows, n_cols = x.shape
    return pl.pallas_call(
        _body,
        in_specs=[pl.BlockSpec(memory_space=pl.ANY)],
        out_specs=pl.BlockSpec(memory_space=pl.ANY),
        out_shape=jax.ShapeDtypeStruct((n_rows, n_cols), x.dtype),
        scratch_shapes=[
            pltpu.VMEM((_NBUF, _CH_ROWS, n_cols), x.dtype),
            pltpu.SemaphoreType.DMA((_NBUF,)),
            pltpu.SemaphoreType.DMA((_NBUF,)),
        ],
        compiler_params=pltpu.CompilerParams(
            vmem_limit_bytes=60 * 1024 * 1024,
        ),
    )(x)


# speedup vs baseline: 1.0164x; 1.0124x over previous
import jax
import jax.numpy as jnp
from jax.experimental import pallas as pl
from jax.experimental.pallas import tpu as pltpu

_CH_ROWS = 2048
_HALF = _CH_ROWS // 2
_NBUF = 3
_PF = 2


def _body(x_hbm, o_hbm, buf, in_sems, out_sems):
    n_rows = x_hbm.shape[0]
    n = n_rows // _CH_ROWS

    def in_copies(i):
        b = i % _NBUF
        return [
            pltpu.make_async_copy(
                x_hbm.at[pl.ds(i * _CH_ROWS + h * _HALF, _HALF), :],
                buf.at[b, pl.ds(h * _HALF, _HALF), :], in_sems.at[b])
            for h in range(2)
        ]

    def out_copies(i):
        b = i % _NBUF
        return [
            pltpu.make_async_copy(
                buf.at[b, pl.ds(h * _HALF, _HALF), :],
                o_hbm.at[pl.ds(i * _CH_ROWS + h * _HALF, _HALF), :],
                out_sems.at[b])
            for h in range(2)
        ]

    def start(copies):
        for c in copies:
            c.start()

    def wait(copies):
        for c in copies:
            c.wait()

    for i in range(_PF):
        start(in_copies(i))

    for i in range(n):
        b = i % _NBUF
        if i + _PF < n:
            if i + _PF >= _NBUF:
                wait(out_copies(i + _PF - _NBUF))
            start(in_copies(i + _PF))
        wait(in_copies(i))
        buf[b] = jnp.clip(buf[b], 0.0, 1.0)
        start(out_copies(i))

    for i in range(n - _NBUF, n):
        wait(out_copies(i))


def kernel(x):
    n_rows, n_cols = x.shape
    return pl.pallas_call(
        _body,
        in_specs=[pl.BlockSpec(memory_space=pl.ANY)],
        out_specs=pl.BlockSpec(memory_space=pl.ANY),
        out_shape=jax.ShapeDtypeStruct((n_rows, n_cols), x.dtype),
        scratch_shapes=[
            pltpu.VMEM((_NBUF, _CH_ROWS, n_cols), x.dtype),
            pltpu.SemaphoreType.DMA((_NBUF,)),
            pltpu.SemaphoreType.DMA((_NBUF,)),
        ],
        compiler_params=pltpu.CompilerParams(
            vmem_limit_bytes=60 * 1024 * 1024,
        ),
    )(x)
